# Initial kernel scaffold; baseline (speedup 1.0000x reference)
import jax
import jax.numpy as jnp
from jax.experimental import pallas as pl


def kernel(x, edge_index, enc_W, enc_b, W0, W1, W2):
    raise NotImplementedError("write your pallas kernel here")


# speedup vs baseline: 4.3325x; 4.3325x over previous
import functools
import jax
import jax.numpy as jnp
from jax import lax
from jax.experimental import pallas as pl
from jax.experimental.pallas import tpu as pltpu
from jax.experimental.pallas import tpu_sc as plsc

N = 10000
NP = 10240
TRASH = 10000
E = 320000
EP = 327680
EBLK = 128
NBLK = EP // EBLK
NTILES = 32
BLK_T = NBLK // NTILES
EDG_T = EP // NTILES
PADLIN = N * N
WE = 144
RB = 512
GRID = NP // RB
STRIPE = NP // 16

_mesh = plsc.VectorSubcoreMesh(
    core_axis_name="c", subcore_axis_name="s", num_cores=2, num_subcores=16)


def _zero_rows(ref, nrows, width):
    z = jnp.zeros((16,), jnp.float32)

    def body(r, _):
        for kk in range(width // 16):
            ref[r, pl.ds(kk * 16, 16)] = z
        return 0

    lax.fori_loop(0, nrows, body, 0)


def _k0_body(slinp, ssrc_o, sdst_o, mdst_o, degtab_o, indtab_o,
             slin_v, ssrc_v, sdst_v, mdst_v, sdeg_v, ones_v, zbuf_v,
             degacc, indacc):
    cid = lax.axis_index("c")
    sid = lax.axis_index("s")
    wid = cid * 16 + sid

    pltpu.sync_copy(slinp.at[pl.ds(wid * EDG_T, EDG_T + 16)], slin_v)

    io = lax.iota(jnp.int32, 16)
    onerow = jnp.where(io == 0, 1.0, 0.0).astype(jnp.float32)
    zrow = jnp.zeros((16,), jnp.float32)

    def initrows(r, _):
        ones_v[r, :] = onerow
        return 0
    lax.fori_loop(0, EBLK, initrows, 0)

    def initz(r, _):
        zbuf_v[r, :] = zrow
        return 0
    lax.fori_loop(0, STRIPE, initz, 0)

    degacc_st = degacc.at[pl.ds(sid * STRIPE, STRIPE)]
    indacc_st = indacc.at[pl.ds(sid * STRIPE, STRIPE)]
    pltpu.sync_copy(zbuf_v, degacc_st)
    pltpu.sync_copy(zbuf_v, indacc_st)
    plsc.subcore_barrier()

    def blk(j, _):
        for k in range(8):
            off = 16 + j * EBLK + k * 16
            v = slin_v[pl.ds(off, 16)]
            prev = slin_v[pl.ds(off - 1, 16)]
            s = lax.rem(v, N)
            d = lax.div(v, N)
            head = v != prev
            mm = head & (s != d)
            md = jnp.where(mm, d, TRASH)
            sd = jnp.where(v != PADLIN, s, TRASH)
            ssrc_v[j, pl.ds(k * 16, 16)] = s
            sdst_v[j, pl.ds(k * 16, 16)] = jnp.where(v != PADLIN, d, TRASH)
            mdst_v[j, pl.ds(k * 16, 16)] = md
            sdeg_v[j, pl.ds(k * 16, 16)] = sd
        pltpu.sync_copy(ones_v, degacc.at[sdeg_v.at[j]], add=True)
        pltpu.sync_copy(ones_v, indacc.at[mdst_v.at[j]], add=True)
        return 0

    lax.fori_loop(0, BLK_T, blk, 0)

    rows = pl.ds(wid * BLK_T, BLK_T)
    pltpu.sync_copy(ssrc_v, ssrc_o.at[rows])
    pltpu.sync_copy(sdst_v, sdst_o.at[rows])
    pltpu.sync_copy(mdst_v, mdst_o.at[rows])

    plsc.subcore_barrier()
    st = pl.ds(sid * STRIPE, STRIPE)
    pltpu.sync_copy(degacc.at[st], degtab_o.at[cid, st])
    pltpu.sync_copy(indacc.at[st], indtab_o.at[cid, st])


_sc_params = pltpu.CompilerParams(use_tc_tiling_on_sc=False)

_k0 = pl.kernel(
    _k0_body,
    compiler_params=_sc_params,
    out_type=(
        jax.ShapeDtypeStruct((NBLK, EBLK), jnp.int32),
        jax.ShapeDtypeStruct((NBLK, EBLK), jnp.int32),
        jax.ShapeDtypeStruct((NBLK, EBLK), jnp.int32),
        jax.ShapeDtypeStruct((2, NP, 16), jnp.float32),
        jax.ShapeDtypeStruct((2, NP, 16), jnp.float32),
    ),
    mesh=_mesh,
    scratch_types=(
        pltpu.VMEM((EDG_T + 16,), jnp.int32),
        pltpu.VMEM((BLK_T, EBLK), jnp.int32),
        pltpu.VMEM((BLK_T, EBLK), jnp.int32),
        pltpu.VMEM((BLK_T, EBLK), jnp.int32),
        pltpu.VMEM((BLK_T, EBLK), jnp.int32),
        pltpu.VMEM((EBLK, 16), jnp.float32),
        pltpu.VMEM((STRIPE, 16), jnp.float32),
        pltpu.VMEM_SHARED((NP, 16), jnp.float32),
        pltpu.VMEM_SHARED((NP, 16), jnp.float32),
    ),
)


def _rowpass_body(width, tab, gidx, sidx, outpart,
                  gidx_v, sidx_v, rows_v, acc, sem):
    cid = lax.axis_index("c")
    sid = lax.axis_index("s")
    wid = cid * 16 + sid

    rows = pl.ds(wid * BLK_T, BLK_T)
    pltpu.sync_copy(gidx.at[rows], gidx_v)
    pltpu.sync_copy(sidx.at[rows], sidx_v)

    _zero_rows(rows_v, EBLK, width)
    for q in range(STRIPE // EBLK):
        pltpu.sync_copy(rows_v, acc.at[pl.ds(sid * STRIPE + q * EBLK, EBLK)])
    plsc.subcore_barrier()

    def blk(j, _):
        pltpu.async_copy(tab.at[gidx_v.at[j]], rows_v, sem).wait()
        pltpu.sync_copy(rows_v, acc.at[sidx_v.at[j]], add=True)
        return 0

    lax.fori_loop(0, BLK_T, blk, 0)

    plsc.subcore_barrier()
    st = pl.ds(sid * STRIPE, STRIPE)
    pltpu.sync_copy(acc.at[st], outpart.at[cid, st])


def _make_rowpass(width):
    return pl.kernel(
        functools.partial(_rowpass_body, width),
        compiler_params=_sc_params,
        out_type=jax.ShapeDtypeStruct((2, NP, width), jnp.float32),
        mesh=_mesh,
        scratch_types=(
            pltpu.VMEM((BLK_T, EBLK), jnp.int32),
            pltpu.VMEM((BLK_T, EBLK), jnp.int32),
            pltpu.VMEM((EBLK, width), jnp.float32),
            pltpu.VMEM_SHARED((NP, width), jnp.float32),
            pltpu.SemaphoreType.DMA,
        ),
    )


_agg_pass = _make_rowpass(128)
_eng_pass = _make_rowpass(WE)


def _valid_mask(b):
    rid = b * RB + lax.broadcasted_iota(jnp.int32, (RB, 1), 0)
    return rid < N


def _t1_body(xp, encw, encb, degtab, indtab,
             h0_o, hhat_o, dinv_o, dinv2_o, invdeg_o):
    b = pl.program_id(0)
    valid = _valid_mask(b)
    deg_rw = (degtab[0] + degtab[1])[:, 0:1]
    indeg = (indtab[0] + indtab[1])[:, 0:1]
    deg_gcn = indeg + 1.0
    dinv = jnp.where(valid, lax.rsqrt(deg_gcn), 0.0)
    dinv2 = jnp.where(valid, 1.0 / deg_gcn, 0.0)
    invdeg = jnp.where(valid, 1.0 / jnp.maximum(deg_rw, 1.0), 0.0)
    h0 = lax.dot_general(xp[...], encw[...], (((1,), (1,)), ((), ())),
                         preferred_element_type=jnp.float32) + encb[...]
    h0 = jnp.where(valid, h0, 0.0)
    h0_o[...] = h0
    hhat_o[...] = dinv * h0
    dinv_o[...] = jnp.broadcast_to(dinv, (RB, 128))
    dinv2_o[...] = jnp.broadcast_to(dinv2, (RB, 128))
    invdeg_o[...] = jnp.broadcast_to(invdeg, (RB, 128))


_t1 = pl.pallas_call(
    _t1_body,
    grid=(GRID,),
    in_specs=[
        pl.BlockSpec((RB, 128), lambda b: (b, 0)),
        pl.BlockSpec((128, 128), lambda b: (0, 0)),
        pl.BlockSpec((1, 128), lambda b: (0, 0)),
        pl.BlockSpec((2, RB, 16), lambda b: (0, b, 0)),
        pl.BlockSpec((2, RB, 16), lambda b: (0, b, 0)),
    ],
    out_specs=[pl.BlockSpec((RB, 128), lambda b: (b, 0))] * 5,
    out_shape=[jax.ShapeDtypeStruct((NP, 128), jnp.float32)] * 5,
)


def _energy_from_g(hprev, gsum, valid):
    g = gsum[:, :128]
    win = gsum[:, 128:129]
    rv = gsum[:, 129:130]
    sprev = jnp.sum(hprev * hprev, axis=1, keepdims=True)
    vf = valid.astype(jnp.float32)
    return (0.5 * (jnp.sum(rv * vf) + jnp.sum(sprev * win * vf))
            - jnp.sum(hprev * g))


def _t2_body(has_g, *refs):
    if has_g:
        (hprev, apart, dinv, dinv2, invdeg, w, gpart,
         h_o, hhat_o, e_tab_o, n_o, e_o) = refs
    else:
        (hprev, apart, dinv, dinv2, invdeg, w,
         h_o, hhat_o, e_tab_o, n_o) = refs
    b = pl.program_id(0)
    valid = _valid_mask(b)
    a = apart[0] + apart[1]
    u = dinv[...] * a + dinv2[...] * hprev[...]
    hl = lax.dot_general(u, w[...], (((1,), (1,)), ((), ())),
                         preferred_element_type=jnp.float32)
    hl = jnp.where(valid, jnp.maximum(hl, 0.0), 0.0)
    h_o[...] = hl
    hhat_o[...] = dinv[...] * hl
    sl = jnp.sum(hl * hl, axis=1, keepdims=True)
    ic = invdeg[:, 0:1]
    e_tab_o[...] = jnp.concatenate(
        [invdeg[...] * hl, ic, ic * sl, jnp.zeros((RB, WE - 130), jnp.float32)],
        axis=1)

    @pl.when(b == 0)
    def _():
        n_o[...] = jnp.zeros((1, 1), jnp.float32)
        if has_g:
            e_o[...] = jnp.zeros((1, 1), jnp.float32)

    n_o[...] += jnp.sum(hl * hl).reshape(1, 1)
    if has_g:
        e_o[...] += _energy_from_g(
            hprev[...], gpart[0] + gpart[1], valid).reshape(1, 1)


def _make_t2(has_g):
    in_specs = [
        pl.BlockSpec((RB, 128), lambda b: (b, 0)),
        pl.BlockSpec((2, RB, 128), lambda b: (0, b, 0)),
        pl.BlockSpec((RB, 128), lambda b: (b, 0)),
        pl.BlockSpec((RB, 128), lambda b: (b, 0)),
        pl.BlockSpec((RB, 128), lambda b: (b, 0)),
        pl.BlockSpec((128, 128), lambda b: (0, 0)),
    ]
    out_specs = [
        pl.BlockSpec((RB, 128), lambda b: (b, 0)),
        pl.BlockSpec((RB, 128), lambda b: (b, 0)),
        pl.BlockSpec((RB, WE), lambda b: (b, 0)),
        pl.BlockSpec((1, 1), lambda b: (0, 0)),
    ]
    out_shape = [
        jax.ShapeDtypeStruct((NP, 128), jnp.float32),
        jax.ShapeDtypeStruct((NP, 128), jnp.float32),
        jax.ShapeDtypeStruct((NP, WE), jnp.float32),
        jax.ShapeDtypeStruct((1, 1), jnp.float32),
    ]
    if has_g:
        in_specs.append(pl.BlockSpec((2, RB, WE), lambda b: (0, b, 0)))
        out_specs.append(pl.BlockSpec((1, 1), lambda b: (0, 0)))
        out_shape.append(jax.ShapeDtypeStruct((1, 1), jnp.float32))
    return pl.pallas_call(
        functools.partial(_t2_body, has_g),
        grid=(GRID,),
        in_specs=in_specs,
        out_specs=out_specs,
        out_shape=out_shape,
    )


_t2_first = _make_t2(False)
_t2_rest = _make_t2(True)


def _t3_body(hprev, gpart, e_o):
    b = pl.program_id(0)
    valid = _valid_mask(b)

    @pl.when(b == 0)
    def _():
        e_o[...] = jnp.zeros((1, 1), jnp.float32)

    e_o[...] += _energy_from_g(
        hprev[...], gpart[0] + gpart[1], valid).reshape(1, 1)


_t3 = pl.pallas_call(
    _t3_body,
    grid=(GRID,),
    in_specs=[
        pl.BlockSpec((RB, 128), lambda b: (b, 0)),
        pl.BlockSpec((2, RB, WE), lambda b: (0, b, 0)),
    ],
    out_specs=pl.BlockSpec((1, 1), lambda b: (0, 0)),
    out_shape=jax.ShapeDtypeStruct((1, 1), jnp.float32),
)


def kernel(x, edge_index, enc_W, enc_b, W0, W1, W2):
    src = edge_index[0].astype(jnp.int32)
    dst = edge_index[1].astype(jnp.int32)
    lin = dst * N + src
    linp = jnp.concatenate([lin, jnp.full((EP - E,), PADLIN, jnp.int32)])
    slin = jnp.sort(linp)
    slinp = jnp.concatenate([jnp.full((16,), -1, jnp.int32), slin])
    xp = jnp.pad(x, ((0, NP - N), (0, 0)))

    ssrc, sdst, mdst, degtab, indtab = _k0(slinp)
    h0, hhat0, dinvB, dinv2B, invdegB = _t1(
        xp, enc_W, enc_b.reshape(1, 128), degtab, indtab)

    ap1 = _agg_pass(hhat0, ssrc, mdst)
    h1, hhat1, et1, n1 = _t2_first(h0, ap1, dinvB, dinv2B, invdegB, W0)

    gp1 = _eng_pass(et1, ssrc, sdst)
    ap2 = _agg_pass(hhat1, ssrc, mdst)
    h2, hhat2, et2, n2, e1 = _t2_rest(
        h1, ap2, dinvB, dinv2B, invdegB, W1, gp1)

    gp2 = _eng_pass(et2, ssrc, sdst)
    ap3 = _agg_pass(hhat2, ssrc, mdst)
    h3, _, et3, n3, e2 = _t2_rest(
        h2, ap3, dinvB, dinv2B, invdegB, W2, gp2)

    gp3 = _eng_pass(et3, ssrc, sdst)
    e3 = _t3(h3, gp3)

    energies = jnp.stack([e1[0, 0], e2[0, 0], e3[0, 0]])
    norms = jnp.stack([n1[0, 0], n2[0, 0], n3[0, 0]])
    return energies, norms
